# baseline (speedup 1.0000x reference)
import functools

import jax
import jax.numpy as jnp
from jax import lax
from jax.experimental import pallas as pl
from jax.experimental.pallas import tpu as pltpu
from jax.experimental.pallas import tpu_sc as plsc

VOCAB = 100000
EMBED_DIM = 300
EMBED_PAD = 304
BATCH = 4096
HIST = 50

NUM_CORES = 2
NUM_SUBCORES = 16
NW = NUM_CORES * NUM_SUBCORES
B_TOT = BATCH * HIST
PER_W = B_TOT // NW
CHUNK = 64
NCHUNK = PER_W // CHUNK
CWORDS = CHUNK * EMBED_DIM

_PAD_ROWS = 2000

_mesh = plsc.VectorSubcoreMesh(core_axis_name="c", subcore_axis_name="s")


def _pad_body(t_ref, o_ref):
    o_ref[:, :EMBED_DIM] = t_ref[...]
    o_ref[:, EMBED_DIM:] = jnp.zeros((_PAD_ROWS, EMBED_PAD - EMBED_DIM), jnp.float32)


def _tc_pad(table):
    return pl.pallas_call(
        _pad_body,
        grid=(VOCAB // _PAD_ROWS,),
        in_specs=[pl.BlockSpec((_PAD_ROWS, EMBED_DIM), lambda i: (i, 0))],
        out_specs=pl.BlockSpec((_PAD_ROWS, EMBED_PAD), lambda i: (i, 0)),
        out_shape=jax.ShapeDtypeStruct((VOCAB, EMBED_PAD), jnp.float32),
    )(table)


@functools.partial(
    pl.kernel,
    out_type=jax.ShapeDtypeStruct((B_TOT * EMBED_DIM,), jnp.float32),
    mesh=_mesh,
    scratch_types=[
        pltpu.VMEM((NCHUNK, CHUNK), jnp.int32),
        pltpu.VMEM((CHUNK, EMBED_PAD), jnp.float32),
        pltpu.VMEM((CHUNK, EMBED_PAD), jnp.float32),
        pltpu.VMEM((CWORDS + 16,), jnp.float32),
        pltpu.VMEM((CWORDS + 16,), jnp.float32),
        pltpu.SemaphoreType.DMA,
        pltpu.SemaphoreType.DMA,
        pltpu.SemaphoreType.DMA,
        pltpu.SemaphoreType.DMA,
    ],
    compiler_params=pltpu.CompilerParams(
        use_tc_tiling_on_sc=False, needs_layout_passes=False
    ),
)
def _sc_gather(
    idx_hbm, table_hbm, out_hbm,
    idx_v, buf0, buf1, cbuf0, cbuf1, gsem0, gsem1, ssem0, ssem1,
):
    wid = lax.axis_index("s") * NUM_CORES + lax.axis_index("c")
    pltpu.sync_copy(idx_hbm.at[wid], idx_v)
    base = wid * PER_W
    iota = lax.iota(jnp.int32, 16)

    def compact(buf, cbuf):
        def row(j, _):
            dst = iota + 300 * j
            for k in range(19):
                vec = buf[j, pl.ds(16 * k, 16)]
                plsc.store_scatter(cbuf, [dst + 16 * k], vec)
            return 0

        lax.fori_loop(0, CHUNK, row, 0, unroll=2)

    def out_slice(g):
        return out_hbm.at[pl.ds((base + g * CHUNK) * EMBED_DIM, CWORDS)]

    def step(g, buf, cbuf, gsem, ssem):
        pltpu.make_async_copy(table_hbm.at[idx_v.at[g]], buf, gsem).wait()

        @pl.when(g >= 2)
        def _():
            pltpu.make_async_copy(cbuf, out_slice(g - 2), ssem).wait()

        compact(buf, cbuf)

        pltpu.async_copy(cbuf.at[pl.ds(0, CWORDS)], out_slice(g), ssem)

        @pl.when(g + 2 < NCHUNK)
        def _():
            pltpu.async_copy(table_hbm.at[idx_v.at[g + 2]], buf, gsem)

    pltpu.async_copy(table_hbm.at[idx_v.at[0]], buf0, gsem0)
    pltpu.async_copy(table_hbm.at[idx_v.at[1]], buf1, gsem1)

    def body(h, _):
        g = h * 2
        step(g, buf0, cbuf0, gsem0, ssem0)
        step(g + 1, buf1, cbuf1, gsem1, ssem1)
        return 0

    lax.fori_loop(0, NCHUNK // 2, body, 0)

    pltpu.make_async_copy(cbuf0, out_slice(NCHUNK - 2), ssem0).wait()
    pltpu.make_async_copy(cbuf1, out_slice(NCHUNK - 1), ssem1).wait()


def kernel(x, table):
    idx = x.reshape(-1).astype(jnp.int32).reshape(NW, NCHUNK, CHUNK)
    table_p = _tc_pad(table)
    out = _sc_gather(idx, table_p)
    return out.reshape(BATCH, HIST, EMBED_DIM)
